# baseline (speedup 1.0000x reference)
import functools

import jax
import jax.numpy as jnp
from jax import lax
from jax.experimental import pallas as pl
from jax.experimental.pallas import tpu as pltpu
from jax.experimental.pallas import tpu_sc as plsc

B = 16384
D = 128
L = 16
NC = 2
NS = 16
NW = NC * NS
BPW = B // NW
CHUNK = 256
NCHUNK = BPW // CHUNK
GROUPS = CHUNK // L


def _body(users_hbm, items_hbm, ue_hbm, ie_hbm, ub_hbm, ib_hbm, gb_hbm,
          out_hbm, idxu_v, idxi_v, u_v, i_v, ub_v, ib_v, out_v, gb_v, sem):
    wid = lax.axis_index("s") * NC + lax.axis_index("c")
    base = wid * BPW

    pltpu.sync_copy(gb_hbm, gb_v)
    gb = gb_v[...]

    for c in range(NCHUNK):
        off = base + c * CHUNK
        pltpu.sync_copy(users_hbm.at[pl.ds(off, CHUNK)], idxu_v)
        pltpu.sync_copy(items_hbm.at[pl.ds(off, CHUNK)], idxi_v)
        cp_u = pltpu.make_async_copy(ue_hbm.at[idxu_v], u_v, sem)
        cp_i = pltpu.make_async_copy(ie_hbm.at[idxi_v], i_v, sem)
        cp_ub = pltpu.make_async_copy(ub_hbm.at[idxu_v], ub_v, sem)
        cp_ib = pltpu.make_async_copy(ib_hbm.at[idxi_v], ib_v, sem)
        cp_u.start()
        cp_i.start()
        cp_ub.start()
        cp_ib.start()
        cp_u.wait()
        cp_i.wait()
        cp_ub.wait()
        cp_ib.wait()

        def group(g, carry, c=c):
            rows = g * L + lax.broadcasted_iota(jnp.int32, (L,), 0)
            zeros = jnp.zeros((L,), jnp.int32)
            acc = (plsc.load_gather(ub_v, [rows])
                   + plsc.load_gather(ib_v, [rows]) + gb)
            cols = zeros
            one = jnp.ones((L,), jnp.int32)
            for _ in range(D):
                acc = acc + (plsc.load_gather(u_v, [rows, cols])
                             * plsc.load_gather(i_v, [rows, cols]))
                cols = cols + one
            out_v[pl.ds(c * CHUNK + g * L, L)] = acc
            return carry

        lax.fori_loop(0, GROUPS, group, 0, unroll=False)

    pltpu.sync_copy(out_v, out_hbm.at[pl.ds(base, BPW)])


@jax.jit
def _run(users, items, user_emb_w, item_emb_w, user_bias_w, item_bias_w,
         global_bias):
    kern = pl.kernel(
        _body,
        out_type=jax.ShapeDtypeStruct((B,), jnp.float32),
        mesh=plsc.VectorSubcoreMesh(core_axis_name="c", subcore_axis_name="s"),
        scratch_types=[
            pltpu.VMEM((CHUNK,), jnp.int32),
            pltpu.VMEM((CHUNK,), jnp.int32),
            pltpu.VMEM((CHUNK, D), jnp.float32),
            pltpu.VMEM((CHUNK, D), jnp.float32),
            pltpu.VMEM((CHUNK,), jnp.float32),
            pltpu.VMEM((CHUNK,), jnp.float32),
            pltpu.VMEM((BPW,), jnp.float32),
            pltpu.VMEM((L,), jnp.float32),
            pltpu.SemaphoreType.DMA,
        ],
        compiler_params=pltpu.CompilerParams(needs_layout_passes=False),
    )
    gb16 = jnp.full((L,), global_bias, jnp.float32)
    return kern(users, items, user_emb_w, item_emb_w,
                user_bias_w.reshape(-1), item_bias_w.reshape(-1), gb16)


def kernel(users, items, user_emb_w, item_emb_w, user_bias_w, item_bias_w,
           global_bias):
    return _run(users, items, user_emb_w, item_emb_w, user_bias_w,
                item_bias_w, global_bias)


# speedup vs baseline: 1.7725x; 1.7725x over previous
import functools

import jax
import jax.numpy as jnp
from jax import lax
from jax.experimental import pallas as pl
from jax.experimental.pallas import tpu as pltpu
from jax.experimental.pallas import tpu_sc as plsc

B = 16384
D = 128
L = 16
NC = 2
NS = 16
NW = NC * NS
BPW = B // NW
CHUNK = 256
NCHUNK = BPW // CHUNK
GROUPS = CHUNK // L


def _body(users_hbm, items_hbm, ue_hbm, ie_hbm, ub_hbm, ib_hbm, gb_hbm,
          out_hbm, idxu_v, idxi_v, u_v, i_v, ub_v, ib_v, out_v, gb_v, sem):
    wid = lax.axis_index("s") * NC + lax.axis_index("c")
    base = wid * BPW

    pltpu.sync_copy(gb_hbm, gb_v)
    gb = gb_v[...]

    for c in range(NCHUNK):
        off = base + c * CHUNK
        pltpu.sync_copy(users_hbm.at[pl.ds(off, CHUNK)], idxu_v)
        pltpu.sync_copy(items_hbm.at[pl.ds(off, CHUNK)], idxi_v)
        cp_u = pltpu.make_async_copy(ue_hbm.at[idxu_v], u_v, sem)
        cp_i = pltpu.make_async_copy(ie_hbm.at[idxi_v], i_v, sem)
        cp_ub = pltpu.make_async_copy(ub_hbm.at[idxu_v], ub_v, sem)
        cp_ib = pltpu.make_async_copy(ib_hbm.at[idxi_v], ib_v, sem)
        cp_u.start()
        cp_i.start()
        cp_ub.start()
        cp_ib.start()
        cp_u.wait()
        cp_i.wait()
        cp_ub.wait()
        cp_ib.wait()

        def group(g, carry, c=c):
            lane = lax.broadcasted_iota(jnp.int32, (L,), 0)
            rows = g * L + lane
            zeros = jnp.zeros((L,), jnp.int32)
            bias = (plsc.load_gather(ub_v, [rows])
                    + plsc.load_gather(ib_v, [rows]) + gb)
            rowbase = rows * D
            idx = rowbase + lane
            fz = jnp.zeros((L,), jnp.float32)
            accs = [fz, fz, fz, fz]
            for d in range(D):
                u = plsc.load_gather(u_v, [zeros, idx])
                iv = plsc.load_gather(i_v, [zeros, idx])
                accs[d % 4] = accs[d % 4] + u * iv
                if d < D - 1:
                    idx = idx + 1
                    if d >= D - L:
                        wrapped = (idx - rowbase) >= D
                        idx = jnp.where(wrapped, idx - D, idx)
            acc = (accs[0] + accs[1]) + (accs[2] + accs[3]) + bias
            out_v[pl.ds(c * CHUNK + g * L, L)] = acc
            return carry

        lax.fori_loop(0, GROUPS, group, 0, unroll=False)

    pltpu.sync_copy(out_v, out_hbm.at[pl.ds(base, BPW)])


@jax.jit
def _run(users, items, user_emb_w, item_emb_w, user_bias_w, item_bias_w,
         global_bias):
    kern = pl.kernel(
        _body,
        out_type=jax.ShapeDtypeStruct((B,), jnp.float32),
        mesh=plsc.VectorSubcoreMesh(core_axis_name="c", subcore_axis_name="s"),
        scratch_types=[
            pltpu.VMEM((CHUNK,), jnp.int32),
            pltpu.VMEM((CHUNK,), jnp.int32),
            pltpu.VMEM((CHUNK, D), jnp.float32),
            pltpu.VMEM((CHUNK, D), jnp.float32),
            pltpu.VMEM((CHUNK,), jnp.float32),
            pltpu.VMEM((CHUNK,), jnp.float32),
            pltpu.VMEM((BPW,), jnp.float32),
            pltpu.VMEM((L,), jnp.float32),
            pltpu.SemaphoreType.DMA,
        ],
        compiler_params=pltpu.CompilerParams(needs_layout_passes=False),
    )
    gb16 = jnp.full((L,), global_bias, jnp.float32)
    return kern(users, items, user_emb_w, item_emb_w,
                user_bias_w.reshape(-1), item_bias_w.reshape(-1), gb16)


def kernel(users, items, user_emb_w, item_emb_w, user_bias_w, item_bias_w,
           global_bias):
    return _run(users, items, user_emb_w, item_emb_w, user_bias_w,
                item_bias_w, global_bias)
